# Initial kernel scaffold; baseline (speedup 1.0000x reference)
import jax
import jax.numpy as jnp
from jax.experimental import pallas as pl


def kernel(x, padding_mask, bos_emb, eos_emb, pos_table, ln_special_g, ln_special_b, ln_g, ln_b):
    raise NotImplementedError("write your pallas kernel here")


# speedup vs baseline: 3.5123x; 3.5123x over previous
import jax
import jax.numpy as jnp
from jax.experimental import pallas as pl
from jax.experimental.pallas import tpu as pltpu

B, T, C = 16, 2048, 1024
TOUT = T + 2
PADDING_IDX = 1
PRE_SCALE = 7.0
EPS = 1e-5


def _ln(v, g, b):
    m = jnp.mean(v, axis=-1, keepdims=True)
    d = v - m
    var = jnp.mean(d * d, axis=-1, keepdims=True)
    return d * jax.lax.rsqrt(var + EPS) * g + b


def _body(len_ref, x_ref, pos_ref, bos_ref, eos_ref, gsp_ref, bsp_ref,
          g_ref, b_ref, out_ref):
    i = pl.program_id(0)
    lb = len_ref[i]

    gsp = gsp_ref[...]
    bsp = bsp_ref[...]
    bos_n = _ln(bos_ref[...], gsp, bsp)
    eos_n = _ln(eos_ref[...], gsp, bsp)
    g = g_ref[...]
    b = b_ref[...]
    pos_pad = pos_ref[PADDING_IDX:PADDING_IDX + 1, :]

    t = jax.lax.broadcasted_iota(jnp.int32, (T, 1), 0) + 1
    xv = x_ref[...]
    v = jnp.where(t == lb + 1, eos_n * PRE_SCALE, xv * PRE_SCALE)
    p = jnp.where(t <= lb + 1, pos_ref[3:T + 3, :], pos_pad)
    out_ref[1:T + 1, :] = _ln(v + p, g, b)

    out_ref[0:1, :] = _ln(bos_n * PRE_SCALE + pos_ref[2:3, :], g, b)

    full = lb == T
    v_last = jnp.where(full, eos_n * PRE_SCALE, jnp.zeros((1, C), jnp.float32))
    p_last = jnp.where(full, pos_ref[T + 3:T + 4, :], pos_pad)
    out_ref[T + 1:T + 2, :] = _ln(v_last + p_last, g, b)


def kernel(x, padding_mask, bos_emb, eos_emb, pos_table,
           ln_special_g, ln_special_b, ln_g, ln_b):
    lengths0 = (T - jnp.sum(padding_mask.astype(jnp.int32), axis=1)).astype(jnp.int32)

    row = lambda a: a.reshape(1, C)
    out = pl.pallas_call(
        _body,
        grid=(B,),
        in_specs=[
            pl.BlockSpec(memory_space=pltpu.SMEM),
            pl.BlockSpec((None, T, C), lambda i: (i, 0, 0)),
            pl.BlockSpec((pos_table.shape[0], C), lambda i: (0, 0)),
            pl.BlockSpec((1, C), lambda i: (0, 0)),
            pl.BlockSpec((1, C), lambda i: (0, 0)),
            pl.BlockSpec((1, C), lambda i: (0, 0)),
            pl.BlockSpec((1, C), lambda i: (0, 0)),
            pl.BlockSpec((1, C), lambda i: (0, 0)),
            pl.BlockSpec((1, C), lambda i: (0, 0)),
        ],
        out_specs=pl.BlockSpec((None, TOUT, C), lambda i: (i, 0, 0)),
        out_shape=jax.ShapeDtypeStruct((B, TOUT, C), jnp.float32),
    )(lengths0, x, pos_table, row(bos_emb), row(eos_emb),
      row(ln_special_g), row(ln_special_b), row(ln_g), row(ln_b))

    lengths = lengths0 + 2
    new_padding_mask = jnp.arange(TOUT, dtype=jnp.int32)[None, :] >= lengths[:, None]
    return (out, new_padding_mask, lengths)
